# baseline (speedup 1.0000x reference)
import functools

import jax
import jax.numpy as jnp
from jax import lax
from jax.experimental import pallas as pl
from jax.experimental.pallas import tpu as pltpu
from jax.experimental.pallas import tpu_sc as plsc

R = 16384
COLS = 2048
NC = 2
NS = 16
NW = NC * NS
ROWS_W = R // NW
BR = 8
BC = COLS
NCHUNK = ROWS_W // BR
NTRIP = (NCHUNK + 2) // 3
TBL_PAD = 4104

MAGIC_IN = 12615680.0
BIAS = 1262485504


def _build_lut(i, tbl_v, lut_v):
    v = (i * 16 - 32768) + lax.iota(jnp.int32, 16)
    vc = jnp.minimum(jnp.maximum(v, -2048), 2048) + 2048
    s = plsc.load_gather(tbl_v, [vc])
    prod = v * s
    q0 = lax.shift_right_arithmetic(prod, 8)
    r = lax.bitwise_and(prod, 255)
    inc = lax.shift_right_arithmetic(r + lax.bitwise_and(q0, 1) + 127, 8)
    yq = q0 + inc
    lut_v[pl.ds(i * 16, 16)] = yq.astype(jnp.float32) * (1.0 / 256.0)


def _silu_body(x_hbm, tbl_hbm, out_hbm, tbl_v, lut_v,
               x0, x1, x2, si0, si1, si2, so0, so1, so2):
    wid = lax.axis_index("s") * NC + lax.axis_index("c")
    row_base = wid * ROWS_W
    bufs = (x0, x1, x2)
    isems, osems = (si0, si1, si2), (so0, so1, so2)

    def block_slice(ci):
        return pl.ds(row_base + ci * BR, BR)

    for b in range(2):
        pltpu.async_copy(x_hbm.at[block_slice(b)], bufs[b], isems[b])

    pltpu.sync_copy(tbl_hbm, tbl_v)

    zeros = jnp.zeros((16,), jnp.float32)

    @plsc.parallel_loop(0, 1920, unroll=8)
    def _(i):
        lut_v[pl.ds(i * 16, 16)] = zeros

    @plsc.parallel_loop(1920, 2177, unroll=4)
    def _(i):
        _build_lut(i, tbl_v, lut_v)

    @plsc.parallel_loop(2177, 4096, unroll=8)
    def _(i):
        v = (i * 16 - 32768) + lax.iota(jnp.int32, 16)
        lut_v[pl.ds(i * 16, 16)] = v.astype(jnp.float32) * (1.0 / 256.0)

    def do_trip(p, carry):
        for b in range(3):
            ci = p * 3 + b

            @pl.when(ci < NCHUNK)
            def _():
                xb, isem, osem = bufs[b], isems[b], osems[b]
                pltpu.make_async_copy(x_hbm.at[pl.ds(0, BR)], xb, isem).wait()

                for r in range(BR):
                    @plsc.parallel_loop(0, BC // 16, unroll=8)
                    def _(vi):
                        xv = xb[r, pl.ds(vi * 16, 16)]
                        t = xv * 256.0 + MAGIC_IN
                        bits = plsc.bitcast(t, jnp.int32)
                        idx = jnp.minimum(jnp.maximum(bits, BIAS),
                                          BIAS + 65535) - BIAS
                        xb[r, pl.ds(vi * 16, 16)] = plsc.load_gather(
                            lut_v, [idx])

                pltpu.async_copy(xb, out_hbm.at[block_slice(ci)], osem)

                @pl.when(ci + 2 < NCHUNK)
                def _():
                    b2 = (b + 2) % 3

                    @pl.when(ci >= 1)
                    def _():
                        pltpu.make_async_copy(
                            bufs[b2], out_hbm.at[pl.ds(0, BR)],
                            osems[b2]).wait()

                    pltpu.async_copy(x_hbm.at[block_slice(ci + 2)],
                                     bufs[b2], isems[b2])
        return carry

    lax.fori_loop(0, NTRIP, do_trip, 0)

    for ci in range(NCHUNK - 3, NCHUNK):
        b = ci % 3
        pltpu.make_async_copy(bufs[b], out_hbm.at[pl.ds(0, BR)],
                              osems[b]).wait()


@jax.jit
def _silu_sc(x2, tbl32):
    mesh = plsc.VectorSubcoreMesh(core_axis_name="c", subcore_axis_name="s")
    fn = pl.kernel(
        _silu_body,
        mesh=mesh,
        compiler_params=pltpu.CompilerParams(
            needs_layout_passes=False, use_tc_tiling_on_sc=True),
        out_type=jax.ShapeDtypeStruct((R, COLS), jnp.float32),
        scratch_types=[
            pltpu.VMEM((TBL_PAD,), jnp.int32),
            pltpu.VMEM((65536,), jnp.float32),
            pltpu.VMEM((BR, BC), jnp.float32),
            pltpu.VMEM((BR, BC), jnp.float32),
            pltpu.VMEM((BR, BC), jnp.float32),
            pltpu.SemaphoreType.DMA,
            pltpu.SemaphoreType.DMA,
            pltpu.SemaphoreType.DMA,
            pltpu.SemaphoreType.DMA,
            pltpu.SemaphoreType.DMA,
            pltpu.SemaphoreType.DMA,
        ],
    )
    return fn(x2, tbl32)


def kernel(x, table):
    tbl32 = jnp.pad(table.astype(jnp.int32), (0, TBL_PAD - table.shape[0]))
    y = _silu_sc(x.reshape(R, COLS), tbl32)
    return y.reshape(x.shape)
